# Initial kernel scaffold; baseline (speedup 1.0000x reference)
import jax
import jax.numpy as jnp
from jax.experimental import pallas as pl


def kernel(x_querry, x_block, prompt_tokens, key_tokens):
    raise NotImplementedError("write your pallas kernel here")


# speedup vs baseline: 1.4709x; 1.4709x over previous
import functools

import jax
import jax.numpy as jnp
from jax import lax
from jax.experimental import pallas as pl
from jax.experimental.pallas import tpu as pltpu
from jax.experimental.pallas import tpu_sc as plsc

EMB_D = 768
KEY_D = 768
POOL = 100
P_LEN = 8
TOP_K = 5
B = 32
CLS = 50
TOK = 77
PREFIX = 1
MID = P_LEN * TOP_K
ROW_D = P_LEN * EMB_D
IDX_PAD = 128
GATH = 8


def _l2n(x):
    n = jnp.linalg.norm(x, axis=1, keepdims=True)
    return x / jnp.clip(n, 1e-12)


def _topk_body(q_ref, k_ref, idx_ref):
    s = lax.dot_general(
        q_ref[:], k_ref[:], (((1,), (1,)), ((), ())),
        preferred_element_type=jnp.float32,
    )
    col = lax.broadcasted_iota(jnp.int32, s.shape, 1)
    parts = []
    for _ in range(TOP_K):
        m = jnp.max(s, axis=1, keepdims=True)
        amax = jnp.min(jnp.where(s == m, col, POOL), axis=1)
        parts.append(amax[:, None])
        s = jnp.where(col == amax[:, None], -jnp.inf, s)
    parts.append(jnp.zeros((B, IDX_PAD - TOP_K), jnp.int32))
    idx_ref[:] = jnp.concatenate(parts, axis=1)


def _assemble_body(xb_ref, sel_ref, out_ref):
    out_ref[:, 0:PREFIX, :] = xb_ref[:, 0:PREFIX, :]
    out_ref[:, PREFIX:PREFIX + MID, :] = jnp.broadcast_to(
        sel_ref[:], (CLS, MID, EMB_D))
    out_ref[:, PREFIX + MID:, :] = xb_ref[:, PREFIX + MID:, :]


def kernel(x_querry, x_block, prompt_tokens, key_tokens):
    n_k = _l2n(key_tokens)
    q_n = lax.stop_gradient(_l2n(x_querry))
    k_idx = pl.pallas_call(
        _topk_body,
        out_shape=jax.ShapeDtypeStruct((B, IDX_PAD), jnp.int32),
    )(q_n, n_k)

    info = plsc.get_sparse_core_info()
    nc, ns = info.num_cores, info.num_subcores

    mesh = plsc.VectorSubcoreMesh(core_axis_name="c", subcore_axis_name="s")

    @functools.partial(
        pl.kernel,
        out_type=jax.ShapeDtypeStruct((B, GATH, ROW_D), jnp.float32),
        mesh=mesh,
        scratch_types=[
            pltpu.VMEM((GATH,), jnp.int32),
            pltpu.VMEM((GATH, ROW_D), jnp.float32),
            pltpu.SemaphoreType.DMA,
        ],
    )
    def _gather_sel(idx_hbm, prompt_hbm, out_hbm, idx8_v, rows_v, sem):
        b = lax.axis_index("s") * nc + lax.axis_index("c")
        pltpu.sync_copy(idx_hbm.at[b, pl.ds(0, GATH)], idx8_v)
        pltpu.async_copy(prompt_hbm.at[idx8_v], rows_v, sem).wait()
        pltpu.sync_copy(rows_v, out_hbm.at[b])

    sel = _gather_sel(k_idx, prompt_tokens.reshape(POOL, ROW_D))

    out = pl.pallas_call(
        _assemble_body,
        grid=(B,),
        in_specs=[
            pl.BlockSpec((CLS, TOK, EMB_D), lambda b: (0, 0, 0)),
            pl.BlockSpec((1, MID, EMB_D), lambda b: (b, 0, 0)),
        ],
        out_specs=pl.BlockSpec((CLS, TOK, EMB_D), lambda b: (b, 0, 0)),
        out_shape=jax.ShapeDtypeStruct((B * CLS, TOK, EMB_D), jnp.float32),
    )(x_block, sel.reshape(B, GATH * P_LEN, EMB_D))
    return out
